# Initial kernel scaffold; baseline (speedup 1.0000x reference)
import jax
import jax.numpy as jnp
from jax.experimental import pallas as pl


def kernel(features, scores, k):
    raise NotImplementedError("write your pallas kernel here")


# speedup vs baseline: 2.7659x; 2.7659x over previous
import jax
import jax.numpy as jnp
import numpy as np
from jax.experimental import pallas as pl
from jax.experimental.pallas import tpu as pltpu

_K = 20
_TJ = 128
_F = 128
_R = 4


def _topk_gather_kernel(feat_ref, sc_ref, tio_ref, sel_ref, smean_ref, msq_ref):
    nr, ti, tj = msq_ref.shape
    x = feat_ref[...]
    msq_ref[...] = jnp.sum(x * x, axis=3)
    tio = tio_ref[0]
    bigf = jnp.float32(ti * tj)
    one = jnp.float32(1.0)
    zero = jnp.float32(0.0)

    mag4 = jnp.sqrt(msq_ref[...])
    hits4 = jnp.zeros((nr, ti, tj), dtype=jnp.bool_)
    flats = [None] * _K
    for it in range(_K):
        m4 = jnp.max(mag4, axis=(1, 2), keepdims=True)
        hitv4 = mag4 == m4
        flats[it] = jnp.min(jnp.where(hitv4, tio[None], bigf),
                            axis=(1, 2), keepdims=True)
        hits4 = jnp.logical_or(hits4, hitv4)
        mag4 = jnp.where(hitv4, jnp.float32(-1.0), mag4)
    for r in range(nr):
        for it in range(_K):
            flat = flats[it][r, 0, 0].astype(jnp.int32)
            i = flat // tj
            j = flat - i * tj
            sel_ref[r, pl.ds(it, 1), :] = feat_ref[r, pl.ds(i, 1), pl.ds(j, 1), :][0]
    ssum4 = jnp.sum(jnp.where(hits4, sc_ref[...], zero),
                    axis=(1, 2), keepdims=True)
    smean_ref[...] = ssum4 / jnp.float32(_K)

    totals = jnp.sum(jnp.where(hits4, one, zero), axis=(1, 2), keepdims=True)
    for r in range(nr):
        total = totals[r, 0, 0]

        def _exact(r=r):
            mag = jnp.sqrt(msq_ref[r])
            hacc = jnp.zeros((ti, tj), dtype=jnp.bool_)
            for it in range(_K):
                m = jnp.max(mag, axis=(0, 1), keepdims=True)
                fv = jnp.min(jnp.where(mag == m, tio, bigf),
                             axis=(0, 1), keepdims=True)
                hit = tio == fv
                hacc = jnp.logical_or(hacc, hit)
                mag = jnp.where(hit, jnp.float32(-1.0), mag)
                flat = fv[0, 0].astype(jnp.int32)
                i = flat // tj
                j = flat - i * tj
                sel_ref[r, pl.ds(it, 1), :] = (
                    feat_ref[r, pl.ds(i, 1), pl.ds(j, 1), :][0])
            ssum = jnp.sum(jnp.where(hacc, sc_ref[r], zero),
                           axis=(0, 1), keepdims=True)
            smean_ref[r] = ssum / jnp.float32(_K)

        pl.when(total != jnp.float32(_K))(_exact)


@jax.jit
def _run(xr, sc3, tio):
    bc, ti, tj, f = xr.shape
    return pl.pallas_call(
        _topk_gather_kernel,
        grid=(bc // _R,),
        in_specs=[
            pl.BlockSpec((_R, ti, tj, f), lambda b: (b, 0, 0, 0)),
            pl.BlockSpec((_R, ti, tj), lambda b: (b, 0, 0)),
            pl.BlockSpec((1, ti, tj), lambda b: (0, 0, 0)),
        ],
        out_specs=[
            pl.BlockSpec((_R, _K, f), lambda b: (b, 0, 0)),
            pl.BlockSpec((_R, 1, 1), lambda b: (b, 0, 0)),
        ],
        out_shape=[
            jax.ShapeDtypeStruct((bc, _K, f), jnp.float32),
            jax.ShapeDtypeStruct((bc, 1, 1), jnp.float32),
        ],
        scratch_shapes=[pltpu.VMEM((_R, ti, tj), jnp.float32)],
    )(xr, sc3, tio)


def kernel(features, scores, k):
    bc, t, f = features.shape
    half = bc // 2
    ti = t // _TJ
    xr = features.reshape(bc, ti, _TJ, f)
    sc3 = scores.reshape(bc, ti, _TJ)
    tio = jnp.asarray(
        np.arange(t, dtype=np.float32).reshape(1, ti, _TJ))
    sel, smean = _run(xr, sc3, tio)
    smean = smean.reshape(bc, 1)
    return (smean[half:], smean[:half], sel[half:], sel[:half])
